# baseline (speedup 1.0000x reference)
import jax
import jax.numpy as jnp
from jax import lax
from jax.experimental import pallas as pl
from jax.experimental.pallas import tpu as pltpu
from jax.experimental.pallas import tpu_sc as plsc

B, N, K, D = 16, 4096, 20, 64
NC, NS, L = 2, 16, 16
NW = NC * NS
DT = D // 8
PAIRS_PER_W = (B * DT) // NW
NT = N // 128


def _sc_gather(idx_hbm, feat_hbm, out_hbm, slab, idx_v, buf, sems, isems):
    wid = lax.axis_index("s") * NC + lax.axis_index("c")
    rows = [jnp.full((L,), di, jnp.int32) for di in range(8)]

    for p in range(PAIRS_PER_W):
        pair = wid * PAIRS_PER_W + p
        b = pair // DT
        dt = pair % DT
        pltpu.sync_copy(feat_hbm.at[b, dt], slab)
        pltpu.sync_copy(idx_hbm.at[0, b // 8, :, b % 8], idx_v.at[0])

        def k_body(k, _):
            par = lax.rem(k, 2)
            nxt = 1 - par

            @pl.when(k >= 2)
            def _wait():
                pltpu.make_async_copy(
                    buf.at[par], out_hbm.at[b, k - 2, dt], sems.at[par]
                ).wait()

            @pl.when(k + 1 < K)
            def _prefetch():
                pltpu.async_copy(
                    idx_hbm.at[k + 1, b // 8, :, b % 8], idx_v.at[nxt], isems.at[nxt]
                )

            @plsc.parallel_loop(0, NT, unroll=3)
            def nt_body(nt):
                for j in range(8):
                    iv = idx_v[par, nt, pl.ds(j * L, L)]
                    hi = jax.lax.shift_right_logical(iv, 7)
                    lo = jax.lax.bitwise_and(iv, 127)
                    sl = pl.ds(j * L, L)
                    for di in range(8):
                        buf[par, nt, di, sl] = plsc.load_gather(
                            slab, [hi, rows[di], lo]
                        )

            pltpu.async_copy(buf.at[par], out_hbm.at[b, k, dt], sems.at[par])

            @pl.when(k + 1 < K)
            def _wait_prefetch():
                pltpu.make_async_copy(
                    idx_hbm.at[k + 1, b // 8, :, b % 8], idx_v.at[nxt], isems.at[nxt]
                ).wait()

            return _

        lax.fori_loop(0, K, k_body, 0)
        for k in (K - 2, K - 1):
            pltpu.make_async_copy(
                buf.at[k % 2], out_hbm.at[b, k, dt], sems.at[k % 2]
            ).wait()


@jax.jit
def kernel(topk_indices, features):
    idx5 = (
        topk_indices.astype(jnp.int32)
        .reshape(2, 8, NT, 128, K)
        .transpose(4, 0, 2, 1, 3)
    )
    feat5 = (
        features.reshape(B, NT, 128, DT, 8).transpose(0, 3, 1, 4, 2)
    )
    mesh = plsc.VectorSubcoreMesh(core_axis_name="c", subcore_axis_name="s")
    out6 = pl.kernel(
        _sc_gather,
        out_type=jax.ShapeDtypeStruct((B, K, DT, NT, 8, 128), jnp.float32),
        mesh=mesh,
        scratch_types=[
            pltpu.VMEM((NT, 8, 128), jnp.float32),
            pltpu.VMEM((2, NT, 128), jnp.int32),
            pltpu.VMEM((2, NT, 8, 128), jnp.float32),
            pltpu.SemaphoreType.DMA((2,)),
            pltpu.SemaphoreType.DMA((2,)),
        ],
        compiler_params=pltpu.CompilerParams(
            use_tc_tiling_on_sc=False, needs_layout_passes=False
        ),
    )(idx5, feat5)
    return out6.transpose(0, 3, 5, 1, 2, 4).reshape(B, N, K, D)


# speedup vs baseline: 1.2209x; 1.2209x over previous
import jax
import jax.numpy as jnp
from jax import lax
from jax.experimental import pallas as pl
from jax.experimental.pallas import tpu as pltpu
from jax.experimental.pallas import tpu_sc as plsc

B, N, K, D = 16, 4096, 20, 64
NC, NS, L = 2, 16, 16
NW = NC * NS
DT = D // 8
PAIRS_PER_W = (B * DT) // NW
NT = N // 128


def _sc_gather(idx_hbm, feat_hbm, out_hbm, slab, idx_v, buf, sems, isems):
    wid = lax.axis_index("s") * NC + lax.axis_index("c")
    rows = [jnp.full((L,), di, jnp.int32) for di in range(8)]

    for p in range(PAIRS_PER_W):
        pair = wid * PAIRS_PER_W + p
        b = pair // DT
        dt = pair % DT
        pltpu.sync_copy(feat_hbm.at[b, dt], slab)
        pltpu.sync_copy(idx_hbm.at[0, b // 8, :, b % 8], idx_v.at[0])

        def k_body(k, _):
            par = lax.rem(k, 2)
            nxt = 1 - par

            @pl.when(k >= 2)
            def _wait():
                pltpu.make_async_copy(
                    buf.at[par], out_hbm.at[b, k - 2, dt], sems.at[par]
                ).wait()

            @pl.when(k + 1 < K)
            def _prefetch():
                pltpu.async_copy(
                    idx_hbm.at[k + 1, b // 8, :, b % 8], idx_v.at[nxt], isems.at[nxt]
                )

            @plsc.parallel_loop(0, NT, unroll=2)
            def nt_body(nt):
                for j in range(8):
                    iv = idx_v[par, nt, pl.ds(j * L, L)]
                    hi = jax.lax.shift_right_logical(iv, 7)
                    lo = jax.lax.bitwise_and(iv, 127)
                    sl = pl.ds(j * L, L)
                    for di in range(8):
                        buf[par, nt, di, sl] = plsc.load_gather(
                            slab, [hi, rows[di], lo]
                        )

            pltpu.async_copy(buf.at[par], out_hbm.at[b, k, dt], sems.at[par])

            @pl.when(k + 1 < K)
            def _wait_prefetch():
                pltpu.make_async_copy(
                    idx_hbm.at[k + 1, b // 8, :, b % 8], idx_v.at[nxt], isems.at[nxt]
                ).wait()

            return _

        lax.fori_loop(0, K, k_body, 0)
        for k in (K - 2, K - 1):
            pltpu.make_async_copy(
                buf.at[k % 2], out_hbm.at[b, k, dt], sems.at[k % 2]
            ).wait()


@jax.jit
def kernel(topk_indices, features):
    idx5 = (
        topk_indices.astype(jnp.int32)
        .reshape(2, 8, NT, 128, K)
        .transpose(4, 0, 2, 1, 3)
    )
    feat5 = (
        features.reshape(B, NT, 128, DT, 8).transpose(0, 3, 1, 4, 2)
    )
    mesh = plsc.VectorSubcoreMesh(core_axis_name="c", subcore_axis_name="s")
    out6 = pl.kernel(
        _sc_gather,
        out_type=jax.ShapeDtypeStruct((B, K, DT, NT, 8, 128), jnp.float32),
        mesh=mesh,
        scratch_types=[
            pltpu.VMEM((NT, 8, 128), jnp.float32),
            pltpu.VMEM((2, NT, 128), jnp.int32),
            pltpu.VMEM((2, NT, 8, 128), jnp.float32),
            pltpu.SemaphoreType.DMA((2,)),
            pltpu.SemaphoreType.DMA((2,)),
        ],
        compiler_params=pltpu.CompilerParams(
            use_tc_tiling_on_sc=False, needs_layout_passes=False
        ),
    )(idx5, feat5)
    return out6.transpose(0, 3, 5, 1, 2, 4).reshape(B, N, K, D)
